# baseline (speedup 1.0000x reference)
import functools

import jax
import jax.numpy as jnp
from jax.experimental import pallas as pl
from jax.experimental.pallas import tpu as pltpu
from jax.experimental.pallas import tpu_sc as plsc

_TEMP = 0.07
_INVT = 1.0 / _TEMP
_EMA = 0.99
_FMAX = jnp.finfo(jnp.float32).max
_FEPS = jnp.finfo(jnp.float32).eps
_LB = 128
_BI = 4


def _logit_kernel(x_ref, o_ref, *, q, k, b2, bi):
    pid = pl.program_id(0)
    li = jax.lax.broadcasted_iota(jnp.int32, (q, k, b2), 2)
    for j in range(bi):
        x = x_ref[j]
        mq3 = jnp.max(jnp.max(x, axis=2, keepdims=True), axis=1, keepdims=True)
        e = jnp.exp((x - mq3) * _INVT)
        sq = jnp.sum(jnp.sum(e, axis=2), axis=1, keepdims=True)
        mq2 = jnp.max(jnp.max(x, axis=2), axis=1, keepdims=True)
        lse = mq2 * _INVT + jnp.log(sq)
        d = jnp.sum(jnp.where(li == pid * bi + j, x, 0.0), axis=2)
        o_ref[j] = d * _INVT - lse


def _math_kernel(logit_ref, mask_ref, det_ref, conf_ref, tc2_ref, *, b, q, k):
    lg = logit_ref[...]
    mkf = mask_ref[...]
    mk = mkf != 0.0
    det3 = jnp.broadcast_to(det_ref[...][None, :, :], (q, k, b))

    cl = jnp.where(mk, lg, -_FMAX)
    mx = jnp.max(cl, axis=1, keepdims=True)
    e = jnp.exp(cl - mx)
    sm = e / jnp.sum(e, axis=1, keepdims=True)
    conf = jnp.where(mk, sm, 0.0)
    conf_ref[...] = conf

    kio = jax.lax.broadcasted_iota(jnp.int32, (q, k, b), 1)
    cmax = jnp.max(conf, axis=1, keepdims=True)
    amax = jnp.min(jnp.where(conf == cmax, kio, k), axis=1, keepdims=True)
    tcf = jnp.where(mk, (kio == amax).astype(jnp.int32), 0)
    co = jnp.max(det3 * tcf, axis=1, keepdims=True)
    tc2_ref[...] = (co == det3).astype(jnp.float32)


def _bank_kernel(blk_ref, lane_ref, pos_ref, fresh_ref,
                 cblk_ref, tc2_ref, logit_ref, mask_ref,
                 out_ref, pseudo_ref, loss_ref,
                 num_acc, den_acc, *, b, q, k):
    i = pl.program_id(0)
    l = lane_ref[i]
    p = pos_ref[i]

    @pl.when(i == 0)
    def _init():
        num_acc[0, 0] = 0.0
        den_acc[0, 0] = 0.0

    cblk = cblk_ref[...]
    li = jax.lax.broadcasted_iota(jnp.int32, (q, k, _LB), 2)
    oldrow = jnp.sum(jnp.where(li == l, cblk, 0.0), axis=2)
    mrow = mask_ref[0]
    lrow = logit_ref[0]
    ps = jnp.where(mrow != 0.0, oldrow, 0.0)
    pseudo_ref[0] = ps
    num_acc[0, 0] += jnp.sum(ps * lrow)
    den_acc[0, 0] += jnp.sum(mrow[:, 0:1])

    pl_ = jnp.mod(p, _LB)
    col = jnp.where(li == pl_, tc2_ref[...], 0.0)
    tc2l = pltpu.roll(col, jnp.mod(l - pl_, _LB), 2)

    @pl.when(fresh_ref[i] == 1)
    def _first_visit():
        out_ref[...] = cblk

    base = out_ref[...]
    out_ref[...] = jnp.where(li == l, _EMA * base + (1.0 - _EMA) * tc2l, base)

    @pl.when(i == b - 1)
    def _fin():
        loss_ref[0, 0] = -num_acc[0, 0] / (den_acc[0, 0] + _FEPS)


def _sc_table_copy(table):
    rows = table.shape[0]
    info = plsc.get_sparse_core_info()
    nw = info.num_cores * info.num_subcores
    rpw = rows // nw
    mesh = plsc.VectorSubcoreMesh(core_axis_name="c", subcore_axis_name="s")

    @functools.partial(
        pl.kernel,
        out_type=jax.ShapeDtypeStruct(table.shape, table.dtype),
        mesh=mesh,
    )
    def copy_k(t_hbm, o_hbm):
        w = jax.lax.axis_index("s") * info.num_cores + jax.lax.axis_index("c")
        pltpu.sync_copy(t_hbm.at[pl.ds(w * rpw, rpw)],
                        o_hbm.at[pl.ds(w * rpw, rpw)])

    return copy_k(table)


def kernel(output, batch_index, det_labels, x_mask, confidence):
    b, b2, q, k = output.shape
    n = confidence.shape[0]

    out_t = jnp.transpose(output, (0, 2, 3, 1))
    logit = pl.pallas_call(
        functools.partial(_logit_kernel, q=q, k=k, b2=b2, bi=_BI),
        grid=(b // _BI,),
        in_specs=[pl.BlockSpec((_BI, q, k, b2), lambda i: (i, 0, 0, 0))],
        out_specs=pl.BlockSpec((_BI, q, k), lambda i: (i, 0, 0)),
        out_shape=jax.ShapeDtypeStruct((b, q, k), jnp.float32),
    )(out_t)

    idx = batch_index.astype(jnp.int32)
    order = jnp.argsort(idx // _LB)
    idx_s = idx[order]
    blk_s = idx_s // _LB
    lane_s = idx_s % _LB
    pos_s = order.astype(jnp.int32)
    fresh_s = jnp.concatenate(
        [jnp.ones((1,), jnp.int32), (blk_s[1:] != blk_s[:-1]).astype(jnp.int32)])

    conf_t = jnp.transpose(confidence, (1, 2, 0))
    base_t = _sc_table_copy(conf_t.reshape(q * k, n)).reshape(q, k, n)
    logit_t = jnp.transpose(logit, (1, 2, 0))
    mask_f = x_mask.astype(jnp.float32)
    mask_t = jnp.transpose(mask_f, (1, 2, 0))
    det_t = jnp.transpose(det_labels.astype(jnp.int32), (1, 0))

    conf_out_t, tc2_t = pl.pallas_call(
        functools.partial(_math_kernel, b=b, q=q, k=k),
        in_specs=[
            pl.BlockSpec(memory_space=pltpu.VMEM),
            pl.BlockSpec(memory_space=pltpu.VMEM),
            pl.BlockSpec(memory_space=pltpu.VMEM),
        ],
        out_specs=[
            pl.BlockSpec(memory_space=pltpu.VMEM),
            pl.BlockSpec(memory_space=pltpu.VMEM),
        ],
        out_shape=[
            jax.ShapeDtypeStruct((q, k, b), jnp.float32),
            jax.ShapeDtypeStruct((q, k, b), jnp.float32),
        ],
    )(logit_t, mask_t, det_t)

    newconf_t, pseudo, loss11 = pl.pallas_call(
        functools.partial(_bank_kernel, b=b, q=q, k=k),
        grid_spec=pltpu.PrefetchScalarGridSpec(
            num_scalar_prefetch=4,
            grid=(b,),
            in_specs=[
                pl.BlockSpec((q, k, _LB), lambda i, bl, la, po, fr: (0, 0, bl[i])),
                pl.BlockSpec((q, k, _LB), lambda i, bl, la, po, fr: (0, 0, po[i] // _LB)),
                pl.BlockSpec((1, q, k), lambda i, bl, la, po, fr: (po[i], 0, 0)),
                pl.BlockSpec((1, q, k), lambda i, bl, la, po, fr: (po[i], 0, 0)),
            ],
            out_specs=[
                pl.BlockSpec((q, k, _LB), lambda i, bl, la, po, fr: (0, 0, bl[i])),
                pl.BlockSpec((1, q, k), lambda i, bl, la, po, fr: (po[i], 0, 0)),
                pl.BlockSpec(memory_space=pltpu.SMEM),
            ],
            scratch_shapes=[
                pltpu.SMEM((1, 1), jnp.float32),
                pltpu.SMEM((1, 1), jnp.float32),
            ],
        ),
        out_shape=[
            jax.ShapeDtypeStruct((q, k, n), jnp.float32),
            jax.ShapeDtypeStruct((b, q, k), jnp.float32),
            jax.ShapeDtypeStruct((1, 1), jnp.float32),
        ],
        input_output_aliases={4: 0},
    )(blk_s, lane_s, pos_s, fresh_s, base_t, tc2_t, logit, mask_f)

    conf_out = jnp.transpose(conf_out_t, (2, 0, 1))
    new_conf = jnp.transpose(newconf_t, (2, 0, 1))
    return (loss11[0, 0], logit, pseudo, conf_out, new_conf)


# speedup vs baseline: 14.0429x; 14.0429x over previous
import functools

import jax
import jax.numpy as jnp
from jax.experimental import pallas as pl
from jax.experimental.pallas import tpu as pltpu

_TEMP = 0.07
_INVT = 1.0 / _TEMP
_EMA = 0.99
_FMAX = jnp.finfo(jnp.float32).max
_FEPS = jnp.finfo(jnp.float32).eps
_LB = 128
_BI = 4


def _logit_kernel(x_ref, o_ref, *, q, k, b2, bi):
    pid = pl.program_id(0)
    li = jax.lax.broadcasted_iota(jnp.int32, (q, k, b2), 2)
    for j in range(bi):
        x = x_ref[j]
        mq3 = jnp.max(jnp.max(x, axis=2, keepdims=True), axis=1, keepdims=True)
        e = jnp.exp((x - mq3) * _INVT)
        sq = jnp.sum(jnp.sum(e, axis=2), axis=1, keepdims=True)
        mq2 = jnp.max(jnp.max(x, axis=2), axis=1, keepdims=True)
        lse = mq2 * _INVT + jnp.log(sq)
        d = jnp.sum(jnp.where(li == pid * bi + j, x, 0.0), axis=2)
        o_ref[j] = d * _INVT - lse


def _math_kernel(logit_ref, mask_ref, det_ref, conf_ref, tc2_ref, *, b, q, k):
    lg = logit_ref[...]
    mkf = mask_ref[...]
    mk = mkf != 0.0
    det3 = jnp.broadcast_to(det_ref[...][None, :, :], (q, k, b))

    cl = jnp.where(mk, lg, -_FMAX)
    mx = jnp.max(cl, axis=1, keepdims=True)
    e = jnp.exp(cl - mx)
    sm = e / jnp.sum(e, axis=1, keepdims=True)
    conf = jnp.where(mk, sm, 0.0)
    conf_ref[...] = conf

    kio = jax.lax.broadcasted_iota(jnp.int32, (q, k, b), 1)
    cmax = jnp.max(conf, axis=1, keepdims=True)
    amax = jnp.min(jnp.where(conf == cmax, kio, k), axis=1, keepdims=True)
    tcf = jnp.where(mk, (kio == amax).astype(jnp.int32), 0)
    co = jnp.max(det3 * tcf, axis=1, keepdims=True)
    tc2_ref[...] = (co == det3).astype(jnp.float32)


def _bank_kernel(blk_ref, lane_ref, pos_ref, fresh_ref,
                 cblk_ref, tc2_ref, logit_ref, mask_ref,
                 out_ref, pseudo_ref, loss_ref,
                 num_acc, den_acc, *, b, q, k):
    i = pl.program_id(0)
    l = lane_ref[i]
    p = pos_ref[i]

    @pl.when(i == 0)
    def _init():
        num_acc[0, 0] = 0.0
        den_acc[0, 0] = 0.0

    cblk = cblk_ref[...]
    li = jax.lax.broadcasted_iota(jnp.int32, (q, k, _LB), 2)
    oldrow = jnp.sum(jnp.where(li == l, cblk, 0.0), axis=2)
    mrow = mask_ref[0]
    lrow = logit_ref[0]
    ps = jnp.where(mrow != 0.0, oldrow, 0.0)
    pseudo_ref[0] = ps
    num_acc[0, 0] += jnp.sum(ps * lrow)
    den_acc[0, 0] += jnp.sum(mrow[:, 0:1])

    pl_ = jnp.mod(p, _LB)
    col = jnp.where(li == pl_, tc2_ref[...], 0.0)
    tc2l = pltpu.roll(col, jnp.mod(l - pl_, _LB), 2)

    @pl.when(fresh_ref[i] == 1)
    def _first_visit():
        out_ref[...] = cblk

    base = out_ref[...]
    out_ref[...] = jnp.where(li == l, _EMA * base + (1.0 - _EMA) * tc2l, base)

    @pl.when(i == b - 1)
    def _fin():
        loss_ref[0, 0] = -num_acc[0, 0] / (den_acc[0, 0] + _FEPS)


def kernel(output, batch_index, det_labels, x_mask, confidence):
    b, b2, q, k = output.shape
    n = confidence.shape[0]

    out_t = jnp.transpose(output, (0, 2, 3, 1))
    logit = pl.pallas_call(
        functools.partial(_logit_kernel, q=q, k=k, b2=b2, bi=_BI),
        grid=(b // _BI,),
        in_specs=[pl.BlockSpec((_BI, q, k, b2), lambda i: (i, 0, 0, 0))],
        out_specs=pl.BlockSpec((_BI, q, k), lambda i: (i, 0, 0)),
        out_shape=jax.ShapeDtypeStruct((b, q, k), jnp.float32),
    )(out_t)

    idx = batch_index.astype(jnp.int32)
    order = jnp.argsort(idx // _LB)
    idx_s = idx[order]
    blk_s = idx_s // _LB
    lane_s = idx_s % _LB
    pos_s = order.astype(jnp.int32)
    fresh_s = jnp.concatenate(
        [jnp.ones((1,), jnp.int32), (blk_s[1:] != blk_s[:-1]).astype(jnp.int32)])

    conf_t = jnp.transpose(confidence, (1, 2, 0))
    logit_t = jnp.transpose(logit, (1, 2, 0))
    mask_f = x_mask.astype(jnp.float32)
    mask_t = jnp.transpose(mask_f, (1, 2, 0))
    det_t = jnp.transpose(det_labels.astype(jnp.int32), (1, 0))

    conf_out_t, tc2_t = pl.pallas_call(
        functools.partial(_math_kernel, b=b, q=q, k=k),
        in_specs=[
            pl.BlockSpec(memory_space=pltpu.VMEM),
            pl.BlockSpec(memory_space=pltpu.VMEM),
            pl.BlockSpec(memory_space=pltpu.VMEM),
        ],
        out_specs=[
            pl.BlockSpec(memory_space=pltpu.VMEM),
            pl.BlockSpec(memory_space=pltpu.VMEM),
        ],
        out_shape=[
            jax.ShapeDtypeStruct((q, k, b), jnp.float32),
            jax.ShapeDtypeStruct((q, k, b), jnp.float32),
        ],
    )(logit_t, mask_t, det_t)

    newconf_t, pseudo, loss11 = pl.pallas_call(
        functools.partial(_bank_kernel, b=b, q=q, k=k),
        grid_spec=pltpu.PrefetchScalarGridSpec(
            num_scalar_prefetch=4,
            grid=(b,),
            in_specs=[
                pl.BlockSpec((q, k, _LB), lambda i, bl, la, po, fr: (0, 0, bl[i])),
                pl.BlockSpec((q, k, _LB), lambda i, bl, la, po, fr: (0, 0, po[i] // _LB)),
                pl.BlockSpec((1, q, k), lambda i, bl, la, po, fr: (po[i], 0, 0)),
                pl.BlockSpec((1, q, k), lambda i, bl, la, po, fr: (po[i], 0, 0)),
            ],
            out_specs=[
                pl.BlockSpec((q, k, _LB), lambda i, bl, la, po, fr: (0, 0, bl[i])),
                pl.BlockSpec((1, q, k), lambda i, bl, la, po, fr: (po[i], 0, 0)),
                pl.BlockSpec(memory_space=pltpu.SMEM),
            ],
            scratch_shapes=[
                pltpu.SMEM((1, 1), jnp.float32),
                pltpu.SMEM((1, 1), jnp.float32),
            ],
        ),
        out_shape=[
            jax.ShapeDtypeStruct((q, k, n), jnp.float32),
            jax.ShapeDtypeStruct((b, q, k), jnp.float32),
            jax.ShapeDtypeStruct((1, 1), jnp.float32),
        ],
        input_output_aliases={4: 0},
    )(blk_s, lane_s, pos_s, fresh_s, conf_t, tc2_t, logit, mask_f)

    conf_out = jnp.transpose(conf_out_t, (2, 0, 1))
    new_conf = jnp.transpose(newconf_t, (2, 0, 1))
    return (loss11[0, 0], logit, pseudo, conf_out, new_conf)


# speedup vs baseline: 14.1295x; 1.0062x over previous
import functools

import jax
import jax.numpy as jnp
from jax.experimental import pallas as pl
from jax.experimental.pallas import tpu as pltpu

_TEMP = 0.07
_INVT = 1.0 / _TEMP
_EMA = 0.99
_FMAX = jnp.finfo(jnp.float32).max
_FEPS = jnp.finfo(jnp.float32).eps
_LB = 128
_BI = 8


def _logit_kernel(x_ref, o_ref, *, q, k, b2, bi):
    pid = pl.program_id(0)
    li = jax.lax.broadcasted_iota(jnp.int32, (q, k, b2), 2)
    for j in range(bi):
        x = x_ref[j]
        mq3 = jnp.max(jnp.max(x, axis=2, keepdims=True), axis=1, keepdims=True)
        e = jnp.exp((x - mq3) * _INVT)
        sq = jnp.sum(jnp.sum(e, axis=2), axis=1, keepdims=True)
        mq2 = jnp.max(jnp.max(x, axis=2), axis=1, keepdims=True)
        lse = mq2 * _INVT + jnp.log(sq)
        d = jnp.sum(jnp.where(li == pid * bi + j, x, 0.0), axis=2)
        o_ref[j] = d * _INVT - lse


def _math_kernel(logit_ref, mask_ref, det_ref, conf_ref, tc2_ref, *, b, q, k):
    lg = logit_ref[...]
    mkf = mask_ref[...]
    mk = mkf != 0.0
    det3 = jnp.broadcast_to(det_ref[...][None, :, :], (q, k, b))

    cl = jnp.where(mk, lg, -_FMAX)
    mx = jnp.max(cl, axis=1, keepdims=True)
    e = jnp.exp(cl - mx)
    sm = e / jnp.sum(e, axis=1, keepdims=True)
    conf = jnp.where(mk, sm, 0.0)
    conf_ref[...] = conf

    kio = jax.lax.broadcasted_iota(jnp.int32, (q, k, b), 1)
    cmax = jnp.max(conf, axis=1, keepdims=True)
    amax = jnp.min(jnp.where(conf == cmax, kio, k), axis=1, keepdims=True)
    tcf = jnp.where(mk, (kio == amax).astype(jnp.int32), 0)
    co = jnp.max(det3 * tcf, axis=1, keepdims=True)
    tc2_ref[...] = (co == det3).astype(jnp.float32)


def _bank_kernel(blk_ref, lane_ref, pos_ref, fresh_ref,
                 cblk_ref, tc2_ref, logit_ref, mask_ref,
                 out_ref, pseudo_ref, loss_ref,
                 num_acc, den_acc, *, b, q, k):
    i = pl.program_id(0)
    l = lane_ref[i]
    p = pos_ref[i]

    @pl.when(i == 0)
    def _init():
        num_acc[0, 0] = 0.0
        den_acc[0, 0] = 0.0

    cblk = cblk_ref[...]
    li = jax.lax.broadcasted_iota(jnp.int32, (q, k, _LB), 2)
    oldrow = jnp.sum(jnp.where(li == l, cblk, 0.0), axis=2)
    mrow = mask_ref[0]
    lrow = logit_ref[0]
    ps = jnp.where(mrow != 0.0, oldrow, 0.0)
    pseudo_ref[0] = ps
    num_acc[0, 0] += jnp.sum(ps * lrow)
    den_acc[0, 0] += jnp.sum(mrow[:, 0:1])

    pl_ = jnp.mod(p, _LB)
    col = jnp.where(li == pl_, tc2_ref[...], 0.0)
    tc2l = pltpu.roll(col, jnp.mod(l - pl_, _LB), 2)

    @pl.when(fresh_ref[i] == 1)
    def _first_visit():
        out_ref[...] = cblk

    base = out_ref[...]
    out_ref[...] = jnp.where(li == l, _EMA * base + (1.0 - _EMA) * tc2l, base)

    @pl.when(i == b - 1)
    def _fin():
        loss_ref[0, 0] = -num_acc[0, 0] / (den_acc[0, 0] + _FEPS)


def kernel(output, batch_index, det_labels, x_mask, confidence):
    b, b2, q, k = output.shape
    n = confidence.shape[0]

    out_t = jnp.transpose(output, (0, 2, 3, 1))
    logit = pl.pallas_call(
        functools.partial(_logit_kernel, q=q, k=k, b2=b2, bi=_BI),
        grid=(b // _BI,),
        in_specs=[pl.BlockSpec((_BI, q, k, b2), lambda i: (i, 0, 0, 0))],
        out_specs=pl.BlockSpec((_BI, q, k), lambda i: (i, 0, 0)),
        out_shape=jax.ShapeDtypeStruct((b, q, k), jnp.float32),
    )(out_t)

    idx = batch_index.astype(jnp.int32)
    order = jnp.argsort(idx // _LB)
    idx_s = idx[order]
    blk_s = idx_s // _LB
    lane_s = idx_s % _LB
    pos_s = order.astype(jnp.int32)
    fresh_s = jnp.concatenate(
        [jnp.ones((1,), jnp.int32), (blk_s[1:] != blk_s[:-1]).astype(jnp.int32)])

    conf_t = jnp.transpose(confidence, (1, 2, 0))
    logit_t = jnp.transpose(logit, (1, 2, 0))
    mask_f = x_mask.astype(jnp.float32)
    mask_t = jnp.transpose(mask_f, (1, 2, 0))
    det_t = jnp.transpose(det_labels.astype(jnp.int32), (1, 0))

    conf_out_t, tc2_t = pl.pallas_call(
        functools.partial(_math_kernel, b=b, q=q, k=k),
        in_specs=[
            pl.BlockSpec(memory_space=pltpu.VMEM),
            pl.BlockSpec(memory_space=pltpu.VMEM),
            pl.BlockSpec(memory_space=pltpu.VMEM),
        ],
        out_specs=[
            pl.BlockSpec(memory_space=pltpu.VMEM),
            pl.BlockSpec(memory_space=pltpu.VMEM),
        ],
        out_shape=[
            jax.ShapeDtypeStruct((q, k, b), jnp.float32),
            jax.ShapeDtypeStruct((q, k, b), jnp.float32),
        ],
    )(logit_t, mask_t, det_t)

    newconf_t, pseudo, loss11 = pl.pallas_call(
        functools.partial(_bank_kernel, b=b, q=q, k=k),
        grid_spec=pltpu.PrefetchScalarGridSpec(
            num_scalar_prefetch=4,
            grid=(b,),
            in_specs=[
                pl.BlockSpec((q, k, _LB), lambda i, bl, la, po, fr: (0, 0, bl[i])),
                pl.BlockSpec((q, k, _LB), lambda i, bl, la, po, fr: (0, 0, po[i] // _LB)),
                pl.BlockSpec((1, q, k), lambda i, bl, la, po, fr: (po[i], 0, 0)),
                pl.BlockSpec((1, q, k), lambda i, bl, la, po, fr: (po[i], 0, 0)),
            ],
            out_specs=[
                pl.BlockSpec((q, k, _LB), lambda i, bl, la, po, fr: (0, 0, bl[i])),
                pl.BlockSpec((1, q, k), lambda i, bl, la, po, fr: (po[i], 0, 0)),
                pl.BlockSpec(memory_space=pltpu.SMEM),
            ],
            scratch_shapes=[
                pltpu.SMEM((1, 1), jnp.float32),
                pltpu.SMEM((1, 1), jnp.float32),
            ],
        ),
        out_shape=[
            jax.ShapeDtypeStruct((q, k, n), jnp.float32),
            jax.ShapeDtypeStruct((b, q, k), jnp.float32),
            jax.ShapeDtypeStruct((1, 1), jnp.float32),
        ],
        input_output_aliases={4: 0},
    )(blk_s, lane_s, pos_s, fresh_s, conf_t, tc2_t, logit, mask_f)

    conf_out = jnp.transpose(conf_out_t, (2, 0, 1))
    new_conf = jnp.transpose(newconf_t, (2, 0, 1))
    return (loss11[0, 0], logit, pseudo, conf_out, new_conf)


# speedup vs baseline: 14.4583x; 1.0233x over previous
import functools

import jax
import jax.numpy as jnp
from jax.experimental import pallas as pl
from jax.experimental.pallas import tpu as pltpu

_TEMP = 0.07
_INVT = 1.0 / _TEMP
_EMA = 0.99
_FMAX = jnp.finfo(jnp.float32).max
_FEPS = jnp.finfo(jnp.float32).eps
_LB = 128
_BI = 8


def _logit_kernel(x_ref, o_ref, *, q, k, b2, bi):
    pid = pl.program_id(0)
    li = jax.lax.broadcasted_iota(jnp.int32, (q, k, b2), 2)
    for j in range(bi):
        x = x_ref[j]
        mq3 = jnp.max(jnp.max(x, axis=2, keepdims=True), axis=1, keepdims=True)
        e = jnp.exp((x - mq3) * _INVT)
        sq = jnp.sum(jnp.sum(e, axis=2), axis=1, keepdims=True)
        mq2 = jnp.max(jnp.max(x, axis=2), axis=1, keepdims=True)
        lse = mq2 * _INVT + jnp.log(sq)
        d = jnp.sum(jnp.where(li == pid * bi + j, x, 0.0), axis=2)
        o_ref[j] = d * _INVT - lse


def _math_kernel(logit_ref, mask_ref, det_ref, conf_ref, tc2_ref, *, b, q, k):
    lg = logit_ref[...]
    mkf = mask_ref[...]
    mk = mkf != 0.0
    det3 = jnp.broadcast_to(det_ref[...][None, :, :], (q, k, b))

    cl = jnp.where(mk, lg, -_FMAX)
    mx = jnp.max(cl, axis=1, keepdims=True)
    e = jnp.exp(cl - mx)
    sm = e / jnp.sum(e, axis=1, keepdims=True)
    conf = jnp.where(mk, sm, 0.0)
    conf_ref[...] = conf

    kio = jax.lax.broadcasted_iota(jnp.int32, (q, k, b), 1)
    cmax = jnp.max(conf, axis=1, keepdims=True)
    amax = jnp.min(jnp.where(conf == cmax, kio, k), axis=1, keepdims=True)
    tcf = jnp.where(mk, (kio == amax).astype(jnp.int32), 0)
    co = jnp.max(det3 * tcf, axis=1, keepdims=True)
    tc2_ref[...] = (co == det3).astype(jnp.float32)


def _bank_kernel(blk_ref, lane_ref, pos_ref, fresh_ref,
                 cblk_ref, tc2_ref, logit_ref, mask_ref,
                 out_ref, pseudo_ref, loss_ref,
                 num_acc, den_acc, *, b, q, k):
    i = pl.program_id(0)
    l = lane_ref[i]
    p = pos_ref[i]

    @pl.when(i == 0)
    def _init():
        num_acc[0, 0] = 0.0
        den_acc[0, 0] = 0.0

    cblk = cblk_ref[...]
    li = jax.lax.broadcasted_iota(jnp.int32, (q, k, _LB), 2)
    oldrow = jnp.sum(jnp.where(li == l, cblk, 0.0), axis=2)
    mrow = mask_ref[0]
    lrow = logit_ref[0]
    ps = jnp.where(mrow != 0.0, oldrow, 0.0)
    pseudo_ref[0] = ps
    num_acc[0, 0] += jnp.sum(ps * lrow)
    den_acc[0, 0] += jnp.sum(mrow[:, 0:1])

    li_b = jax.lax.broadcasted_iota(jnp.int32, (q, k, b), 2)
    col = jnp.where(li_b == p, tc2_ref[...], 0.0)
    colh = col[:, :, 0:_LB] + col[:, :, _LB:b]
    tc2l = pltpu.roll(colh, jnp.mod(l - jnp.mod(p, _LB), _LB), 2)

    @pl.when(fresh_ref[i] == 1)
    def _first_visit():
        out_ref[...] = cblk

    base = out_ref[...]
    out_ref[...] = jnp.where(li == l, _EMA * base + (1.0 - _EMA) * tc2l, base)

    @pl.when(i == b - 1)
    def _fin():
        loss_ref[0, 0] = -num_acc[0, 0] / (den_acc[0, 0] + _FEPS)


def kernel(output, batch_index, det_labels, x_mask, confidence):
    b, b2, q, k = output.shape
    n = confidence.shape[0]

    out_t = jnp.transpose(output, (0, 2, 3, 1))
    logit = pl.pallas_call(
        functools.partial(_logit_kernel, q=q, k=k, b2=b2, bi=_BI),
        grid=(b // _BI,),
        in_specs=[pl.BlockSpec((_BI, q, k, b2), lambda i: (i, 0, 0, 0))],
        out_specs=pl.BlockSpec((_BI, q, k), lambda i: (i, 0, 0)),
        out_shape=jax.ShapeDtypeStruct((b, q, k), jnp.float32),
    )(out_t)

    idx = batch_index.astype(jnp.int32)
    order = jnp.argsort(idx // _LB)
    idx_s = idx[order]
    blk_s = idx_s // _LB
    lane_s = idx_s % _LB
    pos_s = order.astype(jnp.int32)
    fresh_s = jnp.concatenate(
        [jnp.ones((1,), jnp.int32), (blk_s[1:] != blk_s[:-1]).astype(jnp.int32)])

    conf_t = jnp.transpose(confidence, (1, 2, 0))
    logit_t = jnp.transpose(logit, (1, 2, 0))
    mask_f = x_mask.astype(jnp.float32)
    mask_t = jnp.transpose(mask_f, (1, 2, 0))
    det_t = jnp.transpose(det_labels.astype(jnp.int32), (1, 0))

    conf_out_t, tc2_t = pl.pallas_call(
        functools.partial(_math_kernel, b=b, q=q, k=k),
        in_specs=[
            pl.BlockSpec(memory_space=pltpu.VMEM),
            pl.BlockSpec(memory_space=pltpu.VMEM),
            pl.BlockSpec(memory_space=pltpu.VMEM),
        ],
        out_specs=[
            pl.BlockSpec(memory_space=pltpu.VMEM),
            pl.BlockSpec(memory_space=pltpu.VMEM),
        ],
        out_shape=[
            jax.ShapeDtypeStruct((q, k, b), jnp.float32),
            jax.ShapeDtypeStruct((q, k, b), jnp.float32),
        ],
    )(logit_t, mask_t, det_t)

    newconf_t, pseudo, loss11 = pl.pallas_call(
        functools.partial(_bank_kernel, b=b, q=q, k=k),
        grid_spec=pltpu.PrefetchScalarGridSpec(
            num_scalar_prefetch=4,
            grid=(b,),
            in_specs=[
                pl.BlockSpec((q, k, _LB), lambda i, bl, la, po, fr: (0, 0, bl[i])),
                pl.BlockSpec(memory_space=pltpu.VMEM),
                pl.BlockSpec((1, q, k), lambda i, bl, la, po, fr: (po[i], 0, 0)),
                pl.BlockSpec((1, q, k), lambda i, bl, la, po, fr: (po[i], 0, 0)),
            ],
            out_specs=[
                pl.BlockSpec((q, k, _LB), lambda i, bl, la, po, fr: (0, 0, bl[i])),
                pl.BlockSpec((1, q, k), lambda i, bl, la, po, fr: (po[i], 0, 0)),
                pl.BlockSpec(memory_space=pltpu.SMEM),
            ],
            scratch_shapes=[
                pltpu.SMEM((1, 1), jnp.float32),
                pltpu.SMEM((1, 1), jnp.float32),
            ],
        ),
        out_shape=[
            jax.ShapeDtypeStruct((q, k, n), jnp.float32),
            jax.ShapeDtypeStruct((b, q, k), jnp.float32),
            jax.ShapeDtypeStruct((1, 1), jnp.float32),
        ],
        input_output_aliases={4: 0},
    )(blk_s, lane_s, pos_s, fresh_s, conf_t, tc2_t, logit, mask_f)

    conf_out = jnp.transpose(conf_out_t, (2, 0, 1))
    new_conf = jnp.transpose(newconf_t, (2, 0, 1))
    return (loss11[0, 0], logit, pseudo, conf_out, new_conf)
